# baseline (speedup 1.0000x reference)
import jax
import jax.numpy as jnp
from jax import lax
from jax.experimental import pallas as pl
from jax.experimental.pallas import tpu as pltpu
from jax.experimental.pallas import tpu_sc as plsc

_N = 10000
_E = 320000
_F = 128
_H = 128
_T = 8

_NTILES = 16
_NCORES = 2
_TPC = _T // _NCORES
_STRIPE = 640
_NP = _NTILES * _STRIPE
_C = 40
_NCHUNK = 512
_EPT = _C * _NCHUNK
_EPAD = _NTILES * _EPT
_TRASH = _NP - 1
_NSEG = 16
_SCHUNK = _NCHUNK // _NSEG


_DC = 128
_DNCHUNK = _EPT // _DC
_DSEG = 5
_DCHUNK = _DNCHUNK // _DSEG


def _sc_degree_body(dst_hbm, deg_hbm, acc, dst_ids, ones_v, zeros_v):
    c = lax.axis_index("c")
    s = lax.axis_index("s")
    for i in range(_DC // 16):
        ones_v[pl.ds(i * 16, 16)] = jnp.ones((16,), jnp.float32)
    for i in range(_STRIPE // 16):
        zeros_v[pl.ds(i * 16, 16)] = jnp.zeros((16,), jnp.float32)
    for tl in range(_TPC):
        t = c * _TPC + tl
        pltpu.sync_copy(zeros_v, acc.at[pl.ds(s * _STRIPE, _STRIPE)])
        plsc.subcore_barrier()
        for seg in range(_DSEG):
            pltpu.sync_copy(dst_hbm.at[t, s, pl.ds(seg * _DCHUNK, _DCHUNK)],
                            dst_ids)

            def chunk(k, carry):
                pltpu.sync_copy(ones_v, acc.at[dst_ids.at[k]], add=True)
                return carry

            lax.fori_loop(0, _DCHUNK, chunk, 0)
        plsc.subcore_barrier()
        pltpu.sync_copy(acc.at[pl.ds(s * _STRIPE, _STRIPE)],
                        deg_hbm.at[t, pl.ds(s * _STRIPE, _STRIPE)])
        plsc.subcore_barrier()


def _sc_conv_body(xw_hbm, src_hbm, dst_hbm, out_hbm, acc, src_ids, dst_ids,
                  rows0, rows1, rows2, rows3,
                  gsem0, gsem1, gsem2, gsem3,
                  ssem0, ssem1, ssem2, ssem3):
    c = lax.axis_index("c")
    s = lax.axis_index("s")
    bufs = (rows0, rows1, rows2, rows3)
    gsems = (gsem0, gsem1, gsem2, gsem3)
    ssems = (ssem0, ssem1, ssem2, ssem3)

    def t_body(tl, carry):
        t = c * _TPC + tl

        def zrow(r, carry2):
            for j in range(_H // 16):
                rows0[r, pl.ds(j * 16, 16)] = jnp.zeros((16,), jnp.float32)
            return carry2

        lax.fori_loop(0, _C, zrow, 0)

        def zstripe(j, carry2):
            pltpu.sync_copy(rows0, acc.at[pl.ds(s * _STRIPE + j * _C, _C)])
            return carry2

        lax.fori_loop(0, _STRIPE // _C, zstripe, 0)
        plsc.subcore_barrier()

        def seg_body(seg, carry2):
            pltpu.sync_copy(src_hbm.at[t, s, pl.ds(seg * _SCHUNK, _SCHUNK)],
                            src_ids)
            pltpu.sync_copy(dst_hbm.at[t, s, pl.ds(seg * _SCHUNK, _SCHUNK)],
                            dst_ids)
            n = _SCHUNK
            gd = [None] * n
            sd = [None] * n
            for k in range(3):
                gd[k] = pltpu.async_copy(xw_hbm.at[t].at[src_ids.at[k]],
                                         bufs[k % 4], gsems[k % 4])
            for k in range(n):
                if k + 3 < n:
                    if k >= 1:
                        sd[k - 1].wait()
                    gd[k + 3] = pltpu.async_copy(
                        xw_hbm.at[t].at[src_ids.at[k + 3]],
                        bufs[(k + 3) % 4], gsems[(k + 3) % 4])
                gd[k].wait()
                sd[k] = pltpu.async_copy(bufs[k % 4], acc.at[dst_ids.at[k]],
                                         ssems[k % 4], add=True)
            for j in range(n - 4, n):
                sd[j].wait()
            return carry2

        lax.fori_loop(0, _NSEG, seg_body, 0)
        plsc.subcore_barrier()
        pltpu.sync_copy(acc.at[pl.ds(s * _STRIPE, _STRIPE)],
                        out_hbm.at[t, pl.ds(s * _STRIPE, _STRIPE)])
        plsc.subcore_barrier()
        return carry

    lax.fori_loop(0, _TPC, t_body, 0)


_sc_cache = {}


def _build_sc_kernels():
    if "k" in _sc_cache:
        return _sc_cache["k"]
    mesh = plsc.VectorSubcoreMesh(core_axis_name="c", subcore_axis_name="s")
    sc_degree = pl.kernel(
        _sc_degree_body,
        out_type=jax.ShapeDtypeStruct((_T, _NP), jnp.float32),
        mesh=mesh,
        scratch_types=[
            pltpu.VMEM_SHARED((_NP,), jnp.float32),
            pltpu.VMEM((_DCHUNK, _DC), jnp.int32),
            pltpu.VMEM((_DC,), jnp.float32),
            pltpu.VMEM((_STRIPE,), jnp.float32),
        ],
    )
    sc_conv = pl.kernel(
        _sc_conv_body,
        out_type=jax.ShapeDtypeStruct((_T, _NP, _H), jnp.float32),
        mesh=mesh,
        scratch_types=[
            pltpu.VMEM_SHARED((_NP, _H), jnp.float32),
            pltpu.VMEM((_SCHUNK, _C), jnp.int32),
            pltpu.VMEM((_SCHUNK, _C), jnp.int32),
            pltpu.VMEM((_C, _H), jnp.float32),
            pltpu.VMEM((_C, _H), jnp.float32),
            pltpu.VMEM((_C, _H), jnp.float32),
            pltpu.VMEM((_C, _H), jnp.float32),
            pltpu.SemaphoreType.DMA,
            pltpu.SemaphoreType.DMA,
            pltpu.SemaphoreType.DMA,
            pltpu.SemaphoreType.DMA,
            pltpu.SemaphoreType.DMA,
            pltpu.SemaphoreType.DMA,
            pltpu.SemaphoreType.DMA,
            pltpu.SemaphoreType.DMA,
        ],
    )
    _sc_cache["k"] = (sc_degree, sc_conv)
    return _sc_cache["k"]


_BN = 2048


def _tca0_body(x_ref, f_ref, w_ref, o_ref):
    xt = x_ref[0] * f_ref[...]
    o_ref[0] = jnp.dot(xt, w_ref[...], preferred_element_type=jnp.float32)


_tca0 = pl.pallas_call(
    _tca0_body,
    grid=(_T, _NP // _BN),
    in_specs=[
        pl.BlockSpec((1, _BN, _F), lambda t, n: (t, n, 0)),
        pl.BlockSpec((_BN, _F), lambda t, n: (n, 0)),
        pl.BlockSpec((_F, _H), lambda t, n: (0, 0)),
    ],
    out_specs=pl.BlockSpec((1, _BN, _H), lambda t, n: (t, n, 0)),
    out_shape=jax.ShapeDtypeStruct((_T, _NP, _H), jnp.float32),
)


def _tca1_body(xw_ref, deg_ref, o_ref):
    dinv = lax.rsqrt(deg_ref[0] + 1.0)
    o_ref[0] = xw_ref[0] * dinv


_tca1 = pl.pallas_call(
    _tca1_body,
    grid=(_T, _NP // _BN),
    in_specs=[
        pl.BlockSpec((1, _BN, _H), lambda t, n: (t, n, 0)),
        pl.BlockSpec((1, _BN, 1), lambda t, n: (t, n, 0)),
    ],
    out_specs=pl.BlockSpec((1, _BN, _H), lambda t, n: (t, n, 0)),
    out_shape=jax.ShapeDtypeStruct((_T, _NP, _H), jnp.float32),
)


def _tcb_body(a_ref, xw_ref, deg_ref, w_ref, b_ref, o_ref):
    dinv = lax.rsqrt(deg_ref[0] + 1.0)
    out1 = (a_ref[0] + xw_ref[0]) * dinv + b_ref[...]
    xw2 = jnp.dot(out1, w_ref[...], preferred_element_type=jnp.float32)
    o_ref[0] = xw2 * dinv


_tcb = pl.pallas_call(
    _tcb_body,
    grid=(_T, _NP // _BN),
    in_specs=[
        pl.BlockSpec((1, _BN, _H), lambda t, n: (t, n, 0)),
        pl.BlockSpec((1, _BN, _H), lambda t, n: (t, n, 0)),
        pl.BlockSpec((1, _BN, 1), lambda t, n: (t, n, 0)),
        pl.BlockSpec((_H, _H), lambda t, n: (0, 0)),
        pl.BlockSpec((_H,), lambda t, n: (0,)),
    ],
    out_specs=pl.BlockSpec((1, _BN, _H), lambda t, n: (t, n, 0)),
    out_shape=jax.ShapeDtypeStruct((_T, _NP, _H), jnp.float32),
)


_BG = 512


def _gru_cell(gi, gh, h):
    r = jax.nn.sigmoid(gi[:, :_H] + gh[:, :_H])
    z = jax.nn.sigmoid(gi[:, _H:2 * _H] + gh[:, _H:2 * _H])
    n = jnp.tanh(gi[:, 2 * _H:] + r * gh[:, 2 * _H:])
    return (1.0 - z) * n + z * h


def _tcg_body(a_ref, xw_ref, deg_ref, b2_ref,
              wih1_ref, whh1_ref, bih1_ref, bhh1_ref,
              wih2_ref, whh2_ref, bih2_ref, bhh2_ref, o_ref):
    dinv = lax.rsqrt(deg_ref[...] + 1.0)
    seq = (a_ref[...] + xw_ref[...]) * dinv + b2_ref[...]
    seq2 = seq.reshape(_T * _BG, _H)
    gi1 = jnp.dot(seq2, wih1_ref[...],
                  preferred_element_type=jnp.float32) + bih1_ref[...]
    h = jnp.zeros((_BG, _H), jnp.float32)
    h1s = []
    for t in range(_T):
        gh = jnp.dot(h, whh1_ref[...],
                     preferred_element_type=jnp.float32) + bhh1_ref[...]
        h = _gru_cell(gi1[t * _BG:(t + 1) * _BG], gh, h)
        h1s.append(h)
    gi2 = jnp.dot(jnp.concatenate(h1s, axis=0), wih2_ref[...],
                  preferred_element_type=jnp.float32) + bih2_ref[...]
    h = jnp.zeros((_BG, _H), jnp.float32)
    outs = []
    for t in range(_T):
        gh = jnp.dot(h, whh2_ref[...],
                     preferred_element_type=jnp.float32) + bhh2_ref[...]
        h = _gru_cell(gi2[t * _BG:(t + 1) * _BG], gh, h)
        outs.append(h[:, None, :])
    o_ref[...] = jnp.concatenate(outs, axis=1)


_tcg = pl.pallas_call(
    _tcg_body,
    grid=(_NP // _BG,),
    in_specs=[
        pl.BlockSpec((_T, _BG, _H), lambda n: (0, n, 0)),
        pl.BlockSpec((_T, _BG, _H), lambda n: (0, n, 0)),
        pl.BlockSpec((_T, _BG, 1), lambda n: (0, n, 0)),
        pl.BlockSpec((_H,), lambda n: (0,)),
        pl.BlockSpec((_H, 3 * _H), lambda n: (0, 0)),
        pl.BlockSpec((_H, 3 * _H), lambda n: (0, 0)),
        pl.BlockSpec((3 * _H,), lambda n: (0,)),
        pl.BlockSpec((3 * _H,), lambda n: (0,)),
        pl.BlockSpec((_H, 3 * _H), lambda n: (0, 0)),
        pl.BlockSpec((_H, 3 * _H), lambda n: (0, 0)),
        pl.BlockSpec((3 * _H,), lambda n: (0,)),
        pl.BlockSpec((3 * _H,), lambda n: (0,)),
    ],
    out_specs=pl.BlockSpec((_BG, _T, _H), lambda n: (n, 0, 0)),
    out_shape=jax.ShapeDtypeStruct((_NP, _T, _H), jnp.float32),
)


def kernel(x, edge_index, feats, W1, b1, W2, b2,
           Wih1, Whh1, bih1, bhh1, Wih2, Whh2, bih2, bhh2):
    src = edge_index[:, 0, :]
    dst = edge_index[:, 1, :]
    npad = _EPAD - _E
    srcg = jnp.concatenate(
        [src, jnp.zeros((_T, npad), jnp.int32)], axis=1,
    ).reshape(_T, _NTILES, _NCHUNK, _C)
    dst_pad = jnp.concatenate(
        [dst, jnp.full((_T, npad), _TRASH, jnp.int32)], axis=1)
    dst_r = dst_pad.reshape(_T, _NTILES, _NCHUNK, _C)
    dst_d = dst_pad.reshape(_T, _NTILES, _DNCHUNK, _DC)
    x_p = jnp.concatenate(
        [x, jnp.zeros((_T, _NP - _N, _F), jnp.float32)], axis=1)
    feats_p = jnp.concatenate(
        [feats, jnp.zeros((_NP - _N, _F), jnp.float32)], axis=0)

    sc_degree, sc_conv = _build_sc_kernels()
    xw1 = _tca0(x_p, feats_p, W1)
    deg3 = sc_degree(dst_d)[:, :, None]
    xw1s = _tca1(xw1, deg3)
    a1 = sc_conv(xw1s, srcg, dst_r)
    xw2s = _tcb(a1, xw1s, deg3, W2, b1)
    a2 = sc_conv(xw2s, srcg, dst_r)
    out = _tcg(a2, xw2s, deg3, b2,
               Wih1.T, Whh1.T, bih1, bhh1, Wih2.T, Whh2.T, bih2, bhh2)
    return out[:_N]


# speedup vs baseline: 1.0418x; 1.0418x over previous
import jax
import jax.numpy as jnp
from jax import lax
from jax.experimental import pallas as pl
from jax.experimental.pallas import tpu as pltpu
from jax.experimental.pallas import tpu_sc as plsc

_N = 10000
_E = 320000
_F = 128
_H = 128
_T = 8

_NTILES = 16
_NCORES = 2
_TPC = _T // _NCORES
_STRIPE = 640
_NP = _NTILES * _STRIPE
_C = 32
_NCHUNK = 640
_EPT = _C * _NCHUNK
_EPAD = _NTILES * _EPT
_TRASH = _NP - 1
_NSEG = 20
_SCHUNK = _NCHUNK // _NSEG


_DC = 128
_DNCHUNK = _EPT // _DC
_DSEG = 5
_DCHUNK = _DNCHUNK // _DSEG


def _sc_degree_body(dst_hbm, deg_hbm, acc, dst_ids, ones_v, zeros_v):
    c = lax.axis_index("c")
    s = lax.axis_index("s")
    for i in range(_DC // 16):
        ones_v[pl.ds(i * 16, 16)] = jnp.ones((16,), jnp.float32)
    for i in range(_STRIPE // 16):
        zeros_v[pl.ds(i * 16, 16)] = jnp.zeros((16,), jnp.float32)
    for tl in range(_TPC):
        t = c * _TPC + tl
        pltpu.sync_copy(zeros_v, acc.at[pl.ds(s * _STRIPE, _STRIPE)])
        plsc.subcore_barrier()
        for seg in range(_DSEG):
            pltpu.sync_copy(dst_hbm.at[t, s, pl.ds(seg * _DCHUNK, _DCHUNK)],
                            dst_ids)

            def chunk(k, carry):
                pltpu.sync_copy(ones_v, acc.at[dst_ids.at[k]], add=True)
                return carry

            lax.fori_loop(0, _DCHUNK, chunk, 0)
        plsc.subcore_barrier()
        pltpu.sync_copy(acc.at[pl.ds(s * _STRIPE, _STRIPE)],
                        deg_hbm.at[t, pl.ds(s * _STRIPE, _STRIPE)])
        plsc.subcore_barrier()


def _sc_conv_body(xw_hbm, src_hbm, dst_hbm, out_hbm, acc, src_ids, dst_ids,
                  rows0, rows1, rows2, rows3,
                  gsem0, gsem1, gsem2, gsem3,
                  ssem0, ssem1, ssem2, ssem3):
    c = lax.axis_index("c")
    s = lax.axis_index("s")
    bufs = (rows0, rows1, rows2, rows3)
    gsems = (gsem0, gsem1, gsem2, gsem3)
    ssems = (ssem0, ssem1, ssem2, ssem3)

    def t_body(tl, carry):
        t = c * _TPC + tl

        def zrow(r, carry2):
            for j in range(_H // 16):
                rows0[r, pl.ds(j * 16, 16)] = jnp.zeros((16,), jnp.float32)
            return carry2

        lax.fori_loop(0, _C, zrow, 0)

        def zstripe(j, carry2):
            pltpu.sync_copy(rows0, acc.at[pl.ds(s * _STRIPE + j * _C, _C)])
            return carry2

        lax.fori_loop(0, _STRIPE // _C, zstripe, 0)
        plsc.subcore_barrier()

        def seg_body(seg, carry2):
            pltpu.sync_copy(src_hbm.at[t, s, pl.ds(seg * _SCHUNK, _SCHUNK)],
                            src_ids)
            pltpu.sync_copy(dst_hbm.at[t, s, pl.ds(seg * _SCHUNK, _SCHUNK)],
                            dst_ids)
            n = _SCHUNK
            gd = [None] * n
            sd = [None] * n
            for k in range(3):
                gd[k] = pltpu.async_copy(xw_hbm.at[t].at[src_ids.at[k]],
                                         bufs[k % 4], gsems[k % 4])
            for k in range(n):
                if k + 3 < n:
                    if k >= 1:
                        sd[k - 1].wait()
                    gd[k + 3] = pltpu.async_copy(
                        xw_hbm.at[t].at[src_ids.at[k + 3]],
                        bufs[(k + 3) % 4], gsems[(k + 3) % 4])
                gd[k].wait()
                sd[k] = pltpu.async_copy(bufs[k % 4], acc.at[dst_ids.at[k]],
                                         ssems[k % 4], add=True)
            for j in range(n - 4, n):
                sd[j].wait()
            return carry2

        lax.fori_loop(0, _NSEG, seg_body, 0)
        plsc.subcore_barrier()
        pltpu.sync_copy(acc.at[pl.ds(s * _STRIPE, _STRIPE)],
                        out_hbm.at[t, pl.ds(s * _STRIPE, _STRIPE)])
        plsc.subcore_barrier()
        return carry

    lax.fori_loop(0, _TPC, t_body, 0)


_sc_cache = {}


def _build_sc_kernels():
    if "k" in _sc_cache:
        return _sc_cache["k"]
    mesh = plsc.VectorSubcoreMesh(core_axis_name="c", subcore_axis_name="s")
    sc_degree = pl.kernel(
        _sc_degree_body,
        out_type=jax.ShapeDtypeStruct((_T, _NP), jnp.float32),
        mesh=mesh,
        scratch_types=[
            pltpu.VMEM_SHARED((_NP,), jnp.float32),
            pltpu.VMEM((_DCHUNK, _DC), jnp.int32),
            pltpu.VMEM((_DC,), jnp.float32),
            pltpu.VMEM((_STRIPE,), jnp.float32),
        ],
    )
    sc_conv = pl.kernel(
        _sc_conv_body,
        out_type=jax.ShapeDtypeStruct((_T, _NP, _H), jnp.float32),
        mesh=mesh,
        scratch_types=[
            pltpu.VMEM_SHARED((_NP, _H), jnp.float32),
            pltpu.VMEM((_SCHUNK, _C), jnp.int32),
            pltpu.VMEM((_SCHUNK, _C), jnp.int32),
            pltpu.VMEM((_C, _H), jnp.float32),
            pltpu.VMEM((_C, _H), jnp.float32),
            pltpu.VMEM((_C, _H), jnp.float32),
            pltpu.VMEM((_C, _H), jnp.float32),
            pltpu.SemaphoreType.DMA,
            pltpu.SemaphoreType.DMA,
            pltpu.SemaphoreType.DMA,
            pltpu.SemaphoreType.DMA,
            pltpu.SemaphoreType.DMA,
            pltpu.SemaphoreType.DMA,
            pltpu.SemaphoreType.DMA,
            pltpu.SemaphoreType.DMA,
        ],
    )
    _sc_cache["k"] = (sc_degree, sc_conv)
    return _sc_cache["k"]


_BN = 2048


def _tca0_body(x_ref, f_ref, w_ref, o_ref):
    xt = x_ref[0] * f_ref[...]
    o_ref[0] = jnp.dot(xt, w_ref[...], preferred_element_type=jnp.float32)


_tca0 = pl.pallas_call(
    _tca0_body,
    grid=(_T, _NP // _BN),
    in_specs=[
        pl.BlockSpec((1, _BN, _F), lambda t, n: (t, n, 0)),
        pl.BlockSpec((_BN, _F), lambda t, n: (n, 0)),
        pl.BlockSpec((_F, _H), lambda t, n: (0, 0)),
    ],
    out_specs=pl.BlockSpec((1, _BN, _H), lambda t, n: (t, n, 0)),
    out_shape=jax.ShapeDtypeStruct((_T, _NP, _H), jnp.float32),
)


def _tca1_body(xw_ref, deg_ref, o_ref):
    dinv = lax.rsqrt(deg_ref[0] + 1.0)
    o_ref[0] = xw_ref[0] * dinv


_tca1 = pl.pallas_call(
    _tca1_body,
    grid=(_T, _NP // _BN),
    in_specs=[
        pl.BlockSpec((1, _BN, _H), lambda t, n: (t, n, 0)),
        pl.BlockSpec((1, _BN, 1), lambda t, n: (t, n, 0)),
    ],
    out_specs=pl.BlockSpec((1, _BN, _H), lambda t, n: (t, n, 0)),
    out_shape=jax.ShapeDtypeStruct((_T, _NP, _H), jnp.float32),
)


def _tcb_body(a_ref, xw_ref, deg_ref, w_ref, b_ref, o_ref):
    dinv = lax.rsqrt(deg_ref[0] + 1.0)
    out1 = (a_ref[0] + xw_ref[0]) * dinv + b_ref[...]
    xw2 = jnp.dot(out1, w_ref[...], preferred_element_type=jnp.float32)
    o_ref[0] = xw2 * dinv


_tcb = pl.pallas_call(
    _tcb_body,
    grid=(_T, _NP // _BN),
    in_specs=[
        pl.BlockSpec((1, _BN, _H), lambda t, n: (t, n, 0)),
        pl.BlockSpec((1, _BN, _H), lambda t, n: (t, n, 0)),
        pl.BlockSpec((1, _BN, 1), lambda t, n: (t, n, 0)),
        pl.BlockSpec((_H, _H), lambda t, n: (0, 0)),
        pl.BlockSpec((_H,), lambda t, n: (0,)),
    ],
    out_specs=pl.BlockSpec((1, _BN, _H), lambda t, n: (t, n, 0)),
    out_shape=jax.ShapeDtypeStruct((_T, _NP, _H), jnp.float32),
)


_BG = 512


def _gru_cell(gi, gh, h):
    r = jax.nn.sigmoid(gi[:, :_H] + gh[:, :_H])
    z = jax.nn.sigmoid(gi[:, _H:2 * _H] + gh[:, _H:2 * _H])
    n = jnp.tanh(gi[:, 2 * _H:] + r * gh[:, 2 * _H:])
    return (1.0 - z) * n + z * h


def _tcg_body(a_ref, xw_ref, deg_ref, b2_ref,
              wih1_ref, whh1_ref, bih1_ref, bhh1_ref,
              wih2_ref, whh2_ref, bih2_ref, bhh2_ref, o_ref):
    dinv = lax.rsqrt(deg_ref[...] + 1.0)
    seq = (a_ref[...] + xw_ref[...]) * dinv + b2_ref[...]
    seq2 = seq.reshape(_T * _BG, _H)
    gi1 = jnp.dot(seq2, wih1_ref[...],
                  preferred_element_type=jnp.float32) + bih1_ref[...]
    h = jnp.zeros((_BG, _H), jnp.float32)
    h1s = []
    for t in range(_T):
        gh = jnp.dot(h, whh1_ref[...],
                     preferred_element_type=jnp.float32) + bhh1_ref[...]
        h = _gru_cell(gi1[t * _BG:(t + 1) * _BG], gh, h)
        h1s.append(h)
    gi2 = jnp.dot(jnp.concatenate(h1s, axis=0), wih2_ref[...],
                  preferred_element_type=jnp.float32) + bih2_ref[...]
    h = jnp.zeros((_BG, _H), jnp.float32)
    outs = []
    for t in range(_T):
        gh = jnp.dot(h, whh2_ref[...],
                     preferred_element_type=jnp.float32) + bhh2_ref[...]
        h = _gru_cell(gi2[t * _BG:(t + 1) * _BG], gh, h)
        outs.append(h[:, None, :])
    o_ref[...] = jnp.concatenate(outs, axis=1)


_tcg = pl.pallas_call(
    _tcg_body,
    grid=(_NP // _BG,),
    in_specs=[
        pl.BlockSpec((_T, _BG, _H), lambda n: (0, n, 0)),
        pl.BlockSpec((_T, _BG, _H), lambda n: (0, n, 0)),
        pl.BlockSpec((_T, _BG, 1), lambda n: (0, n, 0)),
        pl.BlockSpec((_H,), lambda n: (0,)),
        pl.BlockSpec((_H, 3 * _H), lambda n: (0, 0)),
        pl.BlockSpec((_H, 3 * _H), lambda n: (0, 0)),
        pl.BlockSpec((3 * _H,), lambda n: (0,)),
        pl.BlockSpec((3 * _H,), lambda n: (0,)),
        pl.BlockSpec((_H, 3 * _H), lambda n: (0, 0)),
        pl.BlockSpec((_H, 3 * _H), lambda n: (0, 0)),
        pl.BlockSpec((3 * _H,), lambda n: (0,)),
        pl.BlockSpec((3 * _H,), lambda n: (0,)),
    ],
    out_specs=pl.BlockSpec((_BG, _T, _H), lambda n: (n, 0, 0)),
    out_shape=jax.ShapeDtypeStruct((_NP, _T, _H), jnp.float32),
)


def kernel(x, edge_index, feats, W1, b1, W2, b2,
           Wih1, Whh1, bih1, bhh1, Wih2, Whh2, bih2, bhh2):
    src = edge_index[:, 0, :]
    dst = edge_index[:, 1, :]
    npad = _EPAD - _E
    srcg = jnp.concatenate(
        [src, jnp.zeros((_T, npad), jnp.int32)], axis=1,
    ).reshape(_T, _NTILES, _NCHUNK, _C)
    dst_pad = jnp.concatenate(
        [dst, jnp.full((_T, npad), _TRASH, jnp.int32)], axis=1)
    dst_r = dst_pad.reshape(_T, _NTILES, _NCHUNK, _C)
    dst_d = dst_pad.reshape(_T, _NTILES, _DNCHUNK, _DC)
    x_p = jnp.concatenate(
        [x, jnp.zeros((_T, _NP - _N, _F), jnp.float32)], axis=1)
    feats_p = jnp.concatenate(
        [feats, jnp.zeros((_NP - _N, _F), jnp.float32)], axis=0)

    sc_degree, sc_conv = _build_sc_kernels()
    xw1 = _tca0(x_p, feats_p, W1)
    deg3 = sc_degree(dst_d)[:, :, None]
    xw1s = _tca1(xw1, deg3)
    a1 = sc_conv(xw1s, srcg, dst_r)
    xw2s = _tcb(a1, xw1s, deg3, W2, b1)
    a2 = sc_conv(xw2s, srcg, dst_r)
    out = _tcg(a2, xw2s, deg3, b2,
               Wih1.T, Whh1.T, bih1, bhh1, Wih2.T, Whh2.T, bih2, bhh2)
    return out[:_N]
